# baseline (speedup 1.0000x reference)
import dataclasses
import functools

import jax
import jax.numpy as jnp
from jax import lax
from jax.experimental import pallas as pl
from jax.experimental.pallas import tpu as pltpu
from jax.experimental.pallas import tpu_sc as plsc

C = 32
H = 512
W = 512

NC = 2
NS = 16
NW = NC * NS
L = 16

CHUNK = 256
SUB = 128
NSUB = CHUNK // SUB
GROUPS = CHUNK // L
UNROLL = 8
NBUF = 2


def _compiler_params():
    cp = pltpu.CompilerParams(use_tc_tiling_on_sc=False)
    if "needs_layout_passes" in pltpu.CompilerParams.__dataclass_fields__:
        cp = dataclasses.replace(cp, needs_layout_passes=False)
    return cp


def _set_scratch():
    return [
        pltpu.VMEM((CHUNK,), jnp.float32),
        pltpu.VMEM((CHUNK,), jnp.float32),
        pltpu.VMEM((CHUNK,), jnp.int32),
        pltpu.VMEM((CHUNK,), jnp.int32),
        pltpu.VMEM((CHUNK,), jnp.int32),
        pltpu.VMEM((CHUNK,), jnp.int32),
        pltpu.VMEM((CHUNK,), jnp.float32),
        pltpu.VMEM((CHUNK,), jnp.float32),
        pltpu.VMEM((CHUNK, C), jnp.float32),
        pltpu.VMEM((CHUNK, C), jnp.float32),
        pltpu.VMEM((CHUNK, C), jnp.float32),
        pltpu.VMEM((CHUNK, C), jnp.float32),
        pltpu.VMEM((CHUNK, C), jnp.float32),
    ]


NSET = len(_set_scratch())


@functools.cache
def _make_sc_lookup(n, start, count):
    npw = count // NW
    chunks = npw // CHUNK
    assert chunks >= 4 and chunks % 2 == 0
    mesh = plsc.VectorSubcoreMesh(core_axis_name="c", subcore_axis_name="s")

    @functools.partial(
        pl.kernel,
        out_type=jax.ShapeDtypeStruct((count, C), jnp.float32),
        mesh=mesh,
        compiler_params=_compiler_params(),
        scratch_types=_set_scratch() + _set_scratch() + [
            pltpu.SemaphoreType.DMA,
            pltpu.SemaphoreType.DMA,
            pltpu.SemaphoreType.DMA,
            pltpu.SemaphoreType.DMA,
            pltpu.SemaphoreType.DMA,
            pltpu.SemaphoreType.DMA,
        ],
    )
    def lookup(xs_hbm, ys_hbm, table_hbm, out_hbm, *scr):
        sets = [scr[:NSET], scr[NSET:2 * NSET]]
        sem_g = [scr[2 * NSET], scr[2 * NSET + 1]]
        sem_o = [scr[2 * NSET + 2], scr[2 * NSET + 3]]
        sem_c = [scr[2 * NSET + 4], scr[2 * NSET + 5]]
        wid = lax.axis_index("s") * NC + lax.axis_index("c")
        base = wid * npw

        def bufs(st):
            (xs_v, ys_v, i00_v, i01_v, i10_v, i11_v,
             fx_v, fy_v, t00_v, t01_v, t10_v, t11_v, o_v) = sets[st]
            return (xs_v, ys_v, (i00_v, i01_v, i10_v, i11_v),
                    (fx_v, fy_v), (t00_v, t01_v, t10_v, t11_v), o_v)

        def issue_coords(k, st):
            xs_v, ys_v = bufs(st)[:2]
            off = start + base + k * CHUNK
            pltpu.async_copy(xs_hbm.at[pl.ds(off, CHUNK)], xs_v, sem_c[st])
            pltpu.async_copy(ys_hbm.at[pl.ds(off, CHUNK)], ys_v, sem_c[st])

        def finish_stage(k, st):
            xs_v, ys_v, i_vs, f_vs, t_vs, _ = bufs(st)
            off = start + base + k * CHUNK
            pltpu.make_async_copy(
                xs_hbm.at[pl.ds(off, CHUNK)], xs_v, sem_c[st]).wait()
            pltpu.make_async_copy(
                ys_hbm.at[pl.ds(off, CHUNK)], ys_v, sem_c[st]).wait()

            @pl.loop(0, GROUPS)
            def _pass1(g):
                s = pl.ds(g * L, L)
                ix = (xs_v[s] + 1.0) * 0.5 * (W - 1)
                iy = (ys_v[s] + 1.0) * 0.5 * (H - 1)
                x0 = jnp.minimum(ix.astype(jnp.int32), W - 2)
                y0 = jnp.minimum(iy.astype(jnp.int32), H - 2)
                i00 = y0 * W + x0
                i_vs[0][s] = i00
                i_vs[1][s] = i00 + 1
                i_vs[2][s] = i00 + W
                i_vs[3][s] = i00 + (W + 1)
                f_vs[0][s] = ix - x0.astype(jnp.float32)
                f_vs[1][s] = iy - y0.astype(jnp.float32)

            for t_v, i_v in zip(t_vs, i_vs):
                for u in range(NSUB):
                    sl = pl.ds(u * SUB, SUB)
                    pltpu.async_copy(table_hbm.at[i_v.at[sl]],
                                     t_v.at[sl], sem_g[st])

        def wait_gathers(st):
            _, _, i_vs, _, t_vs, _ = bufs(st)
            for t_v, i_v in zip(t_vs, i_vs):
                for u in range(NSUB):
                    sl = pl.ds(u * SUB, SUB)
                    pltpu.make_async_copy(table_hbm.at[i_v.at[sl]],
                                          t_v.at[sl], sem_g[st]).wait()

        def wait_out(k_prev, st):
            o_v = bufs(st)[5]
            off = base + k_prev * CHUNK
            pltpu.make_async_copy(
                o_v, out_hbm.at[pl.ds(off, CHUNK)], sem_o[st]).wait()

        def pass2_and_emit(k, st):
            _, _, _, f_vs, t_vs, o_v = bufs(st)
            fx_v, fy_v = f_vs
            t00_v, t01_v, t10_v, t11_v = t_vs

            @pl.loop(0, CHUNK, step=UNROLL)
            def _pass2(p0):
                for dp in range(0, UNROLL, 4):
                    pts = tuple(p0 + dp + i for i in range(4))
                    fxs = [plsc.load_gather(fx_v, [jnp.full((L,), 0, jnp.int32) + p])
                           for p in pts]
                    fys = [plsc.load_gather(fy_v, [jnp.full((L,), 0, jnp.int32) + p])
                           for p in pts]
                    lanes = [(p, pl.ds(h * L, L), i, h)
                             for i, p in enumerate(pts)
                             for h in range(C // L)]
                    a = [t00_v[p, s] for p, s, _, _ in lanes]
                    b = [t01_v[p, s] for p, s, _, _ in lanes]
                    c = [t10_v[p, s] for p, s, _, _ in lanes]
                    d = [t11_v[p, s] for p, s, _, _ in lanes]
                    e = [bb - aa for aa, bb in zip(a, b)]
                    f = [dd - cc for cc, dd in zip(c, d)]
                    m = [fxs[i] * ee for (_, _, i, _), ee in zip(lanes, e)]
                    mm = [fxs[i] * ff for (_, _, i, _), ff in zip(lanes, f)]
                    r = [aa + mi for aa, mi in zip(a, m)]
                    t = [cc + mi for cc, mi in zip(c, mm)]
                    g = [tt - rr for rr, tt in zip(r, t)]
                    nn = [fys[i] * gg for (_, _, i, _), gg in zip(lanes, g)]
                    for (p, _, i, h), rr, ni in zip(lanes, r, nn):
                        o_v[p, pl.ds(h * L, L)] = rr + ni

            off = base + k * CHUNK
            pltpu.async_copy(o_v, out_hbm.at[pl.ds(off, CHUNK)],
                             sem_o[st])

        issue_coords(0, 0)
        issue_coords(1, 1)
        finish_stage(0, 0)
        finish_stage(1, 1)

        @pl.loop(0, chunks - 2, step=2)
        def _main(j):
            for st in range(NBUF):
                k = j + st
                wait_gathers(st)
                issue_coords(k + 2, st)

                @pl.when(k >= 2)
                def _():
                    wait_out(k - 2, st)

                pass2_and_emit(k, st)
                finish_stage(k + 2, st)

        for st in range(NBUF):
            k = chunks - 2 + st
            wait_gathers(st)
            wait_out(k - 2, st)
            pass2_and_emit(k, st)
        for st in range(NBUF):
            wait_out(chunks - 2 + st, st)

    return lookup


NSPLIT = 2


def kernel(x, plane):
    lead = x.shape[:-1]
    coords = x.reshape(-1, 2)
    n = coords.shape[0]
    xs = coords[:, 0]
    ys = coords[:, 1]
    table = jnp.transpose(plane[0], (1, 2, 0)).reshape(H * W, C)
    pieces = [(0, 5 * n // 8), (5 * n // 8, 3 * n // 8)]
    outs = [_make_sc_lookup(n, st, ct)(xs, ys, table) for st, ct in pieces]
    out = jnp.concatenate(outs, axis=0) if len(outs) > 1 else outs[0]
    return out.reshape(lead + (C,))


# speedup vs baseline: 1.0423x; 1.0423x over previous
import dataclasses
import functools

import jax
import jax.numpy as jnp
from jax import lax
from jax.experimental import pallas as pl
from jax.experimental.pallas import tpu as pltpu
from jax.experimental.pallas import tpu_sc as plsc

C = 32
H = 512
W = 512

NC = 2
NS = 16
NW = NC * NS
L = 16

CHUNK = 256
SUB = 128
NSUB = CHUNK // SUB
GROUPS = CHUNK // L
UNROLL = 8
NBUF = 2


def _compiler_params():
    cp = pltpu.CompilerParams(use_tc_tiling_on_sc=False)
    if "needs_layout_passes" in pltpu.CompilerParams.__dataclass_fields__:
        cp = dataclasses.replace(cp, needs_layout_passes=False)
    return cp


def _set_scratch():
    return [
        pltpu.VMEM((CHUNK,), jnp.float32),
        pltpu.VMEM((CHUNK,), jnp.float32),
        pltpu.VMEM((CHUNK,), jnp.int32),
        pltpu.VMEM((CHUNK,), jnp.int32),
        pltpu.VMEM((CHUNK,), jnp.int32),
        pltpu.VMEM((CHUNK,), jnp.int32),
        pltpu.VMEM((CHUNK,), jnp.float32),
        pltpu.VMEM((CHUNK,), jnp.float32),
        pltpu.VMEM((CHUNK, C), jnp.float32),
        pltpu.VMEM((CHUNK, C), jnp.float32),
        pltpu.VMEM((CHUNK, C), jnp.float32),
        pltpu.VMEM((CHUNK, C), jnp.float32),
        pltpu.VMEM((CHUNK, C), jnp.float32),
    ]


NSET = len(_set_scratch())


@functools.cache
def _make_sc_lookup(n, start, count):
    npw = count // NW
    chunks = npw // CHUNK
    assert chunks >= 4 and chunks % 2 == 0
    mesh = plsc.VectorSubcoreMesh(core_axis_name="c", subcore_axis_name="s")

    @functools.partial(
        pl.kernel,
        out_type=jax.ShapeDtypeStruct((count, C), jnp.float32),
        mesh=mesh,
        compiler_params=_compiler_params(),
        scratch_types=_set_scratch() + _set_scratch() + [
            pltpu.SemaphoreType.DMA,
            pltpu.SemaphoreType.DMA,
            pltpu.SemaphoreType.DMA,
            pltpu.SemaphoreType.DMA,
            pltpu.SemaphoreType.DMA,
            pltpu.SemaphoreType.DMA,
        ],
    )
    def lookup(xs_hbm, ys_hbm, table_hbm, out_hbm, *scr):
        sets = [scr[:NSET], scr[NSET:2 * NSET]]
        sem_g = [scr[2 * NSET], scr[2 * NSET + 1]]
        sem_o = [scr[2 * NSET + 2], scr[2 * NSET + 3]]
        sem_c = [scr[2 * NSET + 4], scr[2 * NSET + 5]]
        wid = lax.axis_index("s") * NC + lax.axis_index("c")
        base = wid * npw

        def bufs(st):
            (xs_v, ys_v, i00_v, i01_v, i10_v, i11_v,
             fx_v, fy_v, t00_v, t01_v, t10_v, t11_v, o_v) = sets[st]
            return (xs_v, ys_v, (i00_v, i01_v, i10_v, i11_v),
                    (fx_v, fy_v), (t00_v, t01_v, t10_v, t11_v), o_v)

        def issue_coords(k, st):
            xs_v, ys_v = bufs(st)[:2]
            off = start + base + k * CHUNK
            pltpu.async_copy(xs_hbm.at[pl.ds(off, CHUNK)], xs_v, sem_c[st])
            pltpu.async_copy(ys_hbm.at[pl.ds(off, CHUNK)], ys_v, sem_c[st])

        def finish_stage(k, st):
            xs_v, ys_v, i_vs, f_vs, t_vs, _ = bufs(st)
            off = start + base + k * CHUNK
            pltpu.make_async_copy(
                xs_hbm.at[pl.ds(off, CHUNK)], xs_v, sem_c[st]).wait()
            pltpu.make_async_copy(
                ys_hbm.at[pl.ds(off, CHUNK)], ys_v, sem_c[st]).wait()

            @pl.loop(0, GROUPS)
            def _pass1(g):
                s = pl.ds(g * L, L)
                ix = (xs_v[s] + 1.0) * 0.5 * (W - 1)
                iy = (ys_v[s] + 1.0) * 0.5 * (H - 1)
                x0 = jnp.minimum(ix.astype(jnp.int32), W - 2)
                y0 = jnp.minimum(iy.astype(jnp.int32), H - 2)
                i00 = y0 * W + x0
                i_vs[0][s] = i00
                i_vs[1][s] = i00 + 1
                i_vs[2][s] = i00 + W
                i_vs[3][s] = i00 + (W + 1)
                f_vs[0][s] = ix - x0.astype(jnp.float32)
                f_vs[1][s] = iy - y0.astype(jnp.float32)

            for t_v, i_v in zip(t_vs, i_vs):
                for u in range(NSUB):
                    sl = pl.ds(u * SUB, SUB)
                    pltpu.async_copy(table_hbm.at[i_v.at[sl]],
                                     t_v.at[sl], sem_g[st])

        def wait_gathers(st):
            _, _, i_vs, _, t_vs, _ = bufs(st)
            for t_v, i_v in zip(t_vs, i_vs):
                for u in range(NSUB):
                    sl = pl.ds(u * SUB, SUB)
                    pltpu.make_async_copy(table_hbm.at[i_v.at[sl]],
                                          t_v.at[sl], sem_g[st]).wait()

        def wait_out(k_prev, st):
            o_v = bufs(st)[5]
            off = base + k_prev * CHUNK
            pltpu.make_async_copy(
                o_v, out_hbm.at[pl.ds(off, CHUNK)], sem_o[st]).wait()

        def pass2_and_emit(k, st):
            _, _, _, f_vs, t_vs, o_v = bufs(st)
            fx_v, fy_v = f_vs
            t00_v, t01_v, t10_v, t11_v = t_vs

            @pl.loop(0, CHUNK, step=UNROLL)
            def _pass2(p0):
                for dp in range(0, UNROLL, 4):
                    pts = tuple(p0 + dp + i for i in range(4))
                    fxs = [plsc.load_gather(fx_v, [jnp.full((L,), 0, jnp.int32) + p])
                           for p in pts]
                    fys = [plsc.load_gather(fy_v, [jnp.full((L,), 0, jnp.int32) + p])
                           for p in pts]
                    lanes = [(p, pl.ds(h * L, L), i, h)
                             for i, p in enumerate(pts)
                             for h in range(C // L)]
                    a = [t00_v[p, s] for p, s, _, _ in lanes]
                    b = [t01_v[p, s] for p, s, _, _ in lanes]
                    c = [t10_v[p, s] for p, s, _, _ in lanes]
                    d = [t11_v[p, s] for p, s, _, _ in lanes]
                    e = [bb - aa for aa, bb in zip(a, b)]
                    f = [dd - cc for cc, dd in zip(c, d)]
                    m = [fxs[i] * ee for (_, _, i, _), ee in zip(lanes, e)]
                    mm = [fxs[i] * ff for (_, _, i, _), ff in zip(lanes, f)]
                    r = [aa + mi for aa, mi in zip(a, m)]
                    t = [cc + mi for cc, mi in zip(c, mm)]
                    g = [tt - rr for rr, tt in zip(r, t)]
                    nn = [fys[i] * gg for (_, _, i, _), gg in zip(lanes, g)]
                    for (p, _, i, h), rr, ni in zip(lanes, r, nn):
                        o_v[p, pl.ds(h * L, L)] = rr + ni

            off = base + k * CHUNK
            pltpu.async_copy(o_v, out_hbm.at[pl.ds(off, CHUNK)],
                             sem_o[st])

        issue_coords(0, 0)
        issue_coords(1, 1)
        finish_stage(0, 0)
        finish_stage(1, 1)

        @pl.loop(0, chunks - 2, step=2)
        def _main(j):
            for st in range(NBUF):
                k = j + st
                wait_gathers(st)
                issue_coords(k + 2, st)

                @pl.when(k >= 2)
                def _():
                    wait_out(k - 2, st)

                pass2_and_emit(k, st)
                finish_stage(k + 2, st)

        for st in range(NBUF):
            k = chunks - 2 + st
            wait_gathers(st)
            wait_out(k - 2, st)
            pass2_and_emit(k, st)
        for st in range(NBUF):
            wait_out(chunks - 2 + st, st)

    return lookup


NSPLIT = 2


def kernel(x, plane):
    lead = x.shape[:-1]
    coords = x.reshape(-1, 2)
    n = coords.shape[0]
    xs = coords[:, 0]
    ys = coords[:, 1]
    table = jnp.transpose(plane[0], (1, 2, 0)).reshape(H * W, C)
    piece = n // NSPLIT
    outs = [_make_sc_lookup(n, i * piece, piece)(xs, ys, table)
            for i in range(NSPLIT)]
    out = jnp.concatenate(outs, axis=0) if NSPLIT > 1 else outs[0]
    return out.reshape(lead + (C,))
